# Initial kernel scaffold; baseline (speedup 1.0000x reference)
import jax
import jax.numpy as jnp
from jax.experimental import pallas as pl


def kernel(classifications, regressions, anchors, refined_achors, annotations):
    raise NotImplementedError("write your pallas kernel here")


# speedup vs baseline: 1.3119x; 1.3119x over previous
import jax
import jax.numpy as jnp
from jax.experimental import pallas as pl
from jax.experimental.pallas import tpu as pltpu

_ALPHA = 0.25
_BETA = 1.0 / 9
_MD_THRES = 0.5
_NEG_THRES = _MD_THRES - 0.1
_BIG_I32 = 2**30


def _body(T, TA, B, N, C,
          cls_ref, reg_ref, anc_ref, annT_ref,
          out_cls_ref, out_reg_ref,
          gtmax_ref, gtidx_ref, acc_ref):
    b = pl.program_id(0)
    s = pl.program_id(1)
    tile = jax.lax.rem(s, T)

    @pl.when(s == 0)
    def _init_image():
        gtmax_ref[...] = jnp.full((1, N), -1.0, jnp.float32)
        gtidx_ref[...] = jnp.zeros((1, N), jnp.int32)
        acc_ref[0] = 0.0
        acc_ref[1] = 0.0
        acc_ref[2] = 0.0

    @pl.when((s == 0) & (b == 0))
    def _init_batch():
        acc_ref[3] = 0.0
        acc_ref[4] = 0.0

    ann = annT_ref[0]
    gx1r, gy1r = ann[0:1, :], ann[1:2, :]
    gx2r, gy2r = ann[2:3, :], ann[3:4, :]
    gangr, glblr = ann[4:5, :], ann[5:6, :]
    gcx = (gx1r + gx2r) * 0.5
    gcy = (gy1r + gy2r) * 0.5
    gw = gx2r - gx1r
    gh = gy2r - gy1r
    gx1 = gcx - gw * 0.5
    gx2 = gcx + gw * 0.5
    gy1 = gcy - gh * 0.5
    gy2 = gcy + gh * 0.5
    area_g = (gx2 - gx1) * (gy2 - gy1)

    anc = anc_ref[0]
    ax1r, ay1r = anc[:, 0:1], anc[:, 1:2]
    ax2r, ay2r = anc[:, 2:3], anc[:, 3:4]
    aangr = anc[:, 4:5]
    acx = (ax1r + ax2r) * 0.5
    acy = (ay1r + ay2r) * 0.5
    aw = ax2r - ax1r
    ah = ay2r - ay1r
    ax1 = acx - aw * 0.5
    ax2 = acx + aw * 0.5
    ay1 = acy - ah * 0.5
    ay2 = acy + ah * 0.5
    area_a = (ax2 - ax1) * (ay2 - ay1)

    ix1 = jnp.maximum(ax1, gx1)
    iy1 = jnp.maximum(ay1, gy1)
    ix2 = jnp.minimum(ax2, gx2)
    iy2 = jnp.minimum(ay2, gy2)
    iw = jnp.clip(ix2 - ix1, 0.0)
    ih = jnp.clip(iy2 - iy1, 0.0)
    inter = iw * ih
    ua = area_a + area_g - inter
    iou = inter / jnp.maximum(ua, 1e-8)

    rowid = jax.lax.broadcasted_iota(jnp.int32, (TA, N), 0)

    @pl.when(s < T)
    def _pass1():
        colmax = jnp.max(iou, axis=0, keepdims=True)
        colarg = jnp.min(jnp.where(iou == colmax, rowid, _BIG_I32),
                         axis=0, keepdims=True) + tile * TA
        old = gtmax_ref[...]
        upd = colmax > old
        gtmax_ref[...] = jnp.where(upd, colmax, old)
        gtidx_ref[...] = jnp.where(upd, colarg, gtidx_ref[...])

    @pl.when(s >= T)
    def _pass2():
        iou_max = jnp.max(iou, axis=1, keepdims=True)
        gidcol = jax.lax.broadcasted_iota(jnp.int32, (1, N), 1)
        iou_arg = jnp.min(jnp.where(iou == iou_max, gidcol, _BIG_I32),
                          axis=1, keepdims=True)

        gid = rowid[:, 0:1] + tile * TA
        forced = jnp.any((gtidx_ref[...] == gid) & (gtmax_ref[...] < _MD_THRES),
                         axis=1, keepdims=True)
        positive = (iou_max >= _MD_THRES) | forced

        onehot = (iou_arg == gidcol).astype(jnp.float32)
        a_cx = jnp.sum(onehot * gcx, axis=1, keepdims=True)
        a_cy = jnp.sum(onehot * gcy, axis=1, keepdims=True)
        a_w = jnp.sum(onehot * gw, axis=1, keepdims=True)
        a_h = jnp.sum(onehot * gh, axis=1, keepdims=True)
        a_ang = jnp.sum(onehot * gangr, axis=1, keepdims=True)
        lbl = jnp.sum(onehot * glblr, axis=1, keepdims=True).astype(jnp.int32)

        c = jnp.clip(cls_ref[0], 1e-4, 1.0 - 1e-4)
        cid = jax.lax.broadcasted_iota(jnp.int32, (1, C), 1)
        t = jnp.full((TA, C), -1.0, jnp.float32)
        t = jnp.where(iou_max < _NEG_THRES, 0.0, t)
        t = jnp.where(positive, 0.0, t)
        t = jnp.where(positive & (lbl == cid), 1.0, t)
        af = jnp.where(t == 1.0, _ALPHA, 1.0 - _ALPHA)
        fwb = jnp.where(t == 1.0, 1.0 - c, c)
        fw = af * fwb * fwb
        bce = -(t * jnp.log(c + 1e-6) + (1.0 - t) * jnp.log(1.0 - c + 1e-6))
        cl = jnp.where(t != -1.0, fw * bce, 0.0)

        dx = (a_cx - acx) / aw
        dy = (a_cy - acy) / ah
        dwc = jnp.log(a_w / aw)
        dhc = jnp.log(a_h / ah)
        dt = (a_ang - aangr) * 3.141592653589793 / 180.0
        rt = jnp.concatenate([dx, dy, dwc, dhc, dt], axis=1)
        d = jnp.abs(reg_ref[0] - rt)
        rl = jnp.where(d < _BETA, 0.5 * d * d / _BETA, d - 0.5 * _BETA)
        rl = jnp.where(positive, rl, 0.0)

        acc_ref[0] = acc_ref[0] + jnp.sum(cl)
        acc_ref[1] = acc_ref[1] + jnp.sum(rl)
        acc_ref[2] = acc_ref[2] + jnp.sum(positive.astype(jnp.float32))

    @pl.when(s == 2 * T - 1)
    def _finish_image():
        npos = acc_ref[2]
        den = jnp.maximum(npos, 1.0)
        acc_ref[3] = acc_ref[3] + acc_ref[0] / den
        acc_ref[4] = acc_ref[4] + jnp.where(npos > 0.0,
                                            acc_ref[1] / (den * 5.0), 0.0)

    @pl.when((s == 2 * T - 1) & (b == B - 1))
    def _write_out():
        out_cls_ref[...] = jnp.full((1, 1), acc_ref[3] / B, jnp.float32)
        out_reg_ref[...] = jnp.full((1, 1), acc_ref[4] / B, jnp.float32)


def _pick_tile(A):
    for ta in (2000, 1600, 1024, 800, 512, 400, 256, 200, 160, 128, 80, 64, 40, 32, 16, 8):
        if A % ta == 0:
            return ta
    return A


def kernel(classifications, regressions, anchors, refined_achors, annotations):
    del refined_achors
    B, A, C = classifications.shape
    N = annotations.shape[1]
    TA = _pick_tile(A)
    T = A // TA

    annT = jnp.transpose(annotations, (0, 2, 1))

    import functools
    body = functools.partial(_body, T, TA, B, N, C)
    out_cls, out_reg = pl.pallas_call(
        body,
        grid=(B, 2 * T),
        in_specs=[
            pl.BlockSpec((1, TA, C), lambda b, s: (b, jnp.maximum(s - T, 0), 0)),
            pl.BlockSpec((1, TA, 5), lambda b, s: (b, jnp.maximum(s - T, 0), 0)),
            pl.BlockSpec((1, TA, 5), lambda b, s: (b, jax.lax.rem(s, T), 0)),
            pl.BlockSpec((1, 6, N), lambda b, s: (b, 0, 0)),
        ],
        out_specs=[
            pl.BlockSpec((1, 1), lambda b, s: (0, 0)),
            pl.BlockSpec((1, 1), lambda b, s: (0, 0)),
        ],
        out_shape=[
            jax.ShapeDtypeStruct((1, 1), jnp.float32),
            jax.ShapeDtypeStruct((1, 1), jnp.float32),
        ],
        scratch_shapes=[
            pltpu.VMEM((1, N), jnp.float32),
            pltpu.VMEM((1, N), jnp.int32),
            pltpu.SMEM((8,), jnp.float32),
        ],
    )(classifications, regressions, anchors, annT)
    return (out_cls.reshape(1), out_reg.reshape(1))
